# Initial kernel scaffold; baseline (speedup 1.0000x reference)
import jax
import jax.numpy as jnp
from jax.experimental import pallas as pl


def kernel(embedding, instance_mask):
    raise NotImplementedError("write your pallas kernel here")


# speedup vs baseline: 11.3622x; 11.3622x over previous
import functools

import jax
import jax.numpy as jnp
from jax.experimental import pallas as pl
from jax.experimental.pallas import tpu as pltpu

_DELTA_VAR = 0.5
_DELTA_DIST = 1.5
_ALPHA = 1.0
_BETA = 1.0
_GAMMA = 0.001
_C = 32
_EPS = 1e-12

_HI = jax.lax.Precision.HIGHEST


def _body(nchunks, emb_ref, mask_ref, out_ref,
          sums_ref, cent_ref, cn2c_ref, cntc_ref, cntr_ref, hs_ref, acc_ref):
    b = pl.program_id(0)
    ph = pl.program_id(1)
    n = pl.program_id(2)
    nb = pl.num_programs(0)

    x = emb_ref[0]
    m = mask_ref[0]
    C = _C
    Nc = x.shape[1]
    iota_c = jax.lax.broadcasted_iota(jnp.int32, (C, Nc), 0)
    oh = (m == iota_c).astype(jnp.float32)

    @pl.when((b == 0) & (ph == 0) & (n == 0))
    def _init_acc():
        acc_ref[0] = 0.0
        acc_ref[1] = 0.0
        acc_ref[2] = 0.0
        acc_ref[3] = 0.0

    @pl.when(ph == 0)
    def _phase0():
        @pl.when(n == 0)
        def _z():
            sums_ref[...] = jnp.zeros_like(sums_ref)
            cntc_ref[...] = jnp.zeros_like(cntc_ref)
            cntr_ref[...] = jnp.zeros_like(cntr_ref)

        ones_r = jnp.ones((1, Nc), jnp.float32)
        sums_ref[...] += jax.lax.dot_general(
            oh, x, (((1,), (1,)), ((), ())), precision=_HI)
        cntc_ref[...] += jax.lax.dot_general(
            oh, ones_r, (((1,), (1,)), ((), ())), precision=_HI)
        cntr_ref[...] += jax.lax.dot_general(
            ones_r, oh, (((1,), (1,)), ((), ())), precision=_HI)

        @pl.when(n == nchunks - 1)
        def _centers():
            safe = jnp.maximum(cntc_ref[...], 1.0)
            cent = sums_ref[...] / safe
            cent_ref[...] = cent
            cn2c_ref[...] = jnp.sum(cent * cent, axis=1, keepdims=True)

    @pl.when(ph == 1)
    def _phase1():
        @pl.when(n == 0)
        def _z():
            hs_ref[...] = jnp.zeros_like(hs_ref)

        cent = cent_ref[...]
        p = jax.lax.dot_general(
            cent, x, (((1,), (0,)), ((), ())), precision=_HI)
        pq = p - 0.5 * cn2c_ref[...]
        selq = jnp.sum(oh * pq, axis=0, keepdims=True)
        pix2 = jnp.sum(x * x, axis=0, keepdims=True)
        d = jnp.sqrt(jnp.maximum(pix2 - 2.0 * selq, 0.0) + _EPS)
        hinged = jnp.maximum(d - _DELTA_VAR, 0.0) ** 2
        hs_ref[...] += jax.lax.dot_general(
            oh, hinged, (((1,), (1,)), ((), ())), precision=_HI)

        @pl.when(n == nchunks - 1)
        def _finish():
            cntc = cntc_ref[...]
            safe = jnp.maximum(cntc, 1.0)
            per_inst = hs_ref[...] / safe
            ids_c = jax.lax.broadcasted_iota(jnp.int32, (C, 1), 0)
            validc = ((cntc > 0.0) & (ids_c > 0)).astype(jnp.float32)
            n_inst = jnp.sum(validc)
            lv_b = jnp.sum(validc * per_inst) / jnp.maximum(n_inst, 1.0)

            cent = cent_ref[...]
            g = jax.lax.dot_general(
                cent, cent, (((1,), (1,)), ((), ())), precision=_HI)
            i0 = jax.lax.broadcasted_iota(jnp.int32, (C, C), 0)
            i1 = jax.lax.broadcasted_iota(jnp.int32, (C, C), 1)
            eye = (i0 == i1).astype(jnp.float32)
            cn2r = jnp.sum(g * eye, axis=0, keepdims=True)
            dist2 = jnp.maximum(cn2c_ref[...] + cn2r - 2.0 * g, 0.0)
            dist = jnp.sqrt(dist2 + _EPS)
            cntr = cntr_ref[...]
            ids_r = jax.lax.broadcasted_iota(jnp.int32, (1, C), 1)
            validr = ((cntr > 0.0) & (ids_r > 0)).astype(jnp.float32)
            pairm = validc * validr * (i0 < i1).astype(jnp.float32)
            hd = jnp.maximum(2.0 * _DELTA_DIST - dist, 0.0) ** 2
            npairs = jnp.sum(pairm)
            ld_b = jnp.sum(pairm * hd) / jnp.maximum(npairs, 1.0)

            norms = jnp.sqrt(cn2c_ref[...] + _EPS)
            lr_b = jnp.sum(validc * norms) / jnp.maximum(n_inst, 1.0)

            has = (n_inst > 0.0).astype(jnp.float32)
            acc_ref[0] += lv_b * has
            acc_ref[1] += ld_b * has
            acc_ref[2] += lr_b * has
            acc_ref[3] += has

            @pl.when(b == nb - 1)
            def _emit():
                denom = jnp.maximum(acc_ref[3], 1.0)
                lv = acc_ref[0] / denom
                ld = acc_ref[1] / denom
                lr = acc_ref[2] / denom
                tot = _ALPHA * lv + _BETA * ld + _GAMMA * lr
                lane = jax.lax.broadcasted_iota(jnp.int32, (1, 4), 1)
                out_ref[...] = (
                    tot * (lane == 0) + lv * (lane == 1)
                    + ld * (lane == 2) + lr * (lane == 3)
                ).astype(jnp.float32)


@jax.jit
def kernel(embedding, instance_mask):
    B, E = embedding.shape[0], embedding.shape[1]
    N = embedding.shape[2] * embedding.shape[3]
    emb3 = embedding.reshape(B, E, N)
    mask3 = instance_mask.astype(jnp.int32).reshape(B, 1, N)

    nc = 32768
    nchunks = N // nc

    out = pl.pallas_call(
        functools.partial(_body, nchunks),
        grid=(B, 2, nchunks),
        in_specs=[
            pl.BlockSpec((1, E, nc), lambda b, p, n: (b, 0, n)),
            pl.BlockSpec((1, 1, nc), lambda b, p, n: (b, 0, n)),
        ],
        out_specs=pl.BlockSpec((1, 4), lambda b, p, n: (0, 0)),
        out_shape=jax.ShapeDtypeStruct((1, 4), jnp.float32),
        scratch_shapes=[
            pltpu.VMEM((_C, E), jnp.float32),
            pltpu.VMEM((_C, E), jnp.float32),
            pltpu.VMEM((_C, 1), jnp.float32),
            pltpu.VMEM((_C, 1), jnp.float32),
            pltpu.VMEM((1, _C), jnp.float32),
            pltpu.VMEM((_C, 1), jnp.float32),
            pltpu.SMEM((4,), jnp.float32),
        ],
    )(emb3, mask3)
    return out[0, 0], out[0, 1], out[0, 2], out[0, 3]


# speedup vs baseline: 18.5897x; 1.6361x over previous
import functools

import jax
import jax.numpy as jnp
from jax.experimental import pallas as pl
from jax.experimental.pallas import tpu as pltpu

_DELTA_VAR = 0.5
_DELTA_DIST = 1.5
_ALPHA = 1.0
_BETA = 1.0
_GAMMA = 0.001
_C = 32
_EPS = 1e-12

_HI = jax.lax.Precision.HIGHEST


def _body(nchunks, emb_ref, mask_ref, out_ref,
          sums_ref, cent_ref, cn2c_ref, cntc_ref, hs_ref, acc_ref):
    b = pl.program_id(0)
    ph = pl.program_id(1)
    n = pl.program_id(2)
    nb = pl.num_programs(0)

    x = emb_ref[0]
    m = mask_ref[0]
    C = _C
    Nc = x.shape[1]
    iota_c = jax.lax.broadcasted_iota(jnp.int32, (C, Nc), 0)
    ohb = m == iota_c

    @pl.when((b == 0) & (ph == 0) & (n == 0))
    def _init_acc():
        acc_ref[0] = 0.0
        acc_ref[1] = 0.0
        acc_ref[2] = 0.0
        acc_ref[3] = 0.0

    @pl.when(ph == 0)
    def _phase0():
        @pl.when(n == 0)
        def _z():
            sums_ref[...] = jnp.zeros_like(sums_ref)
            cntc_ref[...] = jnp.zeros_like(cntc_ref)

        oh = ohb.astype(jnp.float32)
        ones_r = jnp.ones((1, Nc), jnp.float32)
        sums_ref[...] += jax.lax.dot_general(
            oh, x, (((1,), (1,)), ((), ())))
        cntc_ref[...] += jax.lax.dot_general(
            oh, ones_r, (((1,), (1,)), ((), ())))

        @pl.when(n == nchunks - 1)
        def _centers():
            safe = jnp.maximum(cntc_ref[...], 1.0)
            cent = sums_ref[...] / safe
            cent_ref[...] = cent
            cn2c_ref[...] = jnp.sum(cent * cent, axis=1, keepdims=True)

    @pl.when(ph == 1)
    def _phase1():
        @pl.when(n == 0)
        def _z():
            hs_ref[...] = jnp.zeros_like(hs_ref)

        cent = cent_ref[...]
        p = jax.lax.dot_general(
            cent, x, (((1,), (0,)), ((), ())))
        pq = p - 0.5 * cn2c_ref[...]
        selq = jnp.sum(jnp.where(ohb, pq, 0.0), axis=0, keepdims=True)
        pix2 = jnp.sum(x * x, axis=0, keepdims=True)
        d = jnp.sqrt(jnp.maximum(pix2 - 2.0 * selq, 0.0) + _EPS)
        hinged = jnp.maximum(d - _DELTA_VAR, 0.0) ** 2
        hs_ref[...] += jax.lax.dot_general(
            ohb.astype(jnp.float32), hinged, (((1,), (1,)), ((), ())))

        @pl.when(n == nchunks - 1)
        def _finish():
            cntc = cntc_ref[...]
            safe = jnp.maximum(cntc, 1.0)
            per_inst = hs_ref[...] / safe
            ids_c = jax.lax.broadcasted_iota(jnp.int32, (C, 1), 0)
            validc = ((cntc > 0.0) & (ids_c > 0)).astype(jnp.float32)
            n_inst = jnp.sum(validc)
            lv_b = jnp.sum(validc * per_inst) / jnp.maximum(n_inst, 1.0)

            cent = cent_ref[...]
            g = jax.lax.dot_general(
                cent, cent, (((1,), (1,)), ((), ())), precision=_HI)
            i0 = jax.lax.broadcasted_iota(jnp.int32, (C, C), 0)
            i1 = jax.lax.broadcasted_iota(jnp.int32, (C, C), 1)
            eye = (i0 == i1).astype(jnp.float32)
            cn2r = jnp.sum(g * eye, axis=0, keepdims=True)
            dist2 = jnp.maximum(cn2c_ref[...] + cn2r - 2.0 * g, 0.0)
            dist = jnp.sqrt(dist2 + _EPS)
            validr = jnp.sum(eye * validc, axis=0, keepdims=True)
            pairm = validc * validr * (i0 < i1).astype(jnp.float32)
            hd = jnp.maximum(2.0 * _DELTA_DIST - dist, 0.0) ** 2
            npairs = jnp.sum(pairm)
            ld_b = jnp.sum(pairm * hd) / jnp.maximum(npairs, 1.0)

            norms = jnp.sqrt(cn2c_ref[...] + _EPS)
            lr_b = jnp.sum(validc * norms) / jnp.maximum(n_inst, 1.0)

            has = (n_inst > 0.0).astype(jnp.float32)
            acc_ref[0] += lv_b * has
            acc_ref[1] += ld_b * has
            acc_ref[2] += lr_b * has
            acc_ref[3] += has

            @pl.when(b == nb - 1)
            def _emit():
                denom = jnp.maximum(acc_ref[3], 1.0)
                lv = acc_ref[0] / denom
                ld = acc_ref[1] / denom
                lr = acc_ref[2] / denom
                tot = _ALPHA * lv + _BETA * ld + _GAMMA * lr
                lane = jax.lax.broadcasted_iota(jnp.int32, (1, 4), 1)
                out_ref[...] = (
                    tot * (lane == 0) + lv * (lane == 1)
                    + ld * (lane == 2) + lr * (lane == 3)
                ).astype(jnp.float32)


@jax.jit
def kernel(embedding, instance_mask):
    B, E = embedding.shape[0], embedding.shape[1]
    N = embedding.shape[2] * embedding.shape[3]
    emb3 = embedding.reshape(B, E, N)
    mask3 = instance_mask.astype(jnp.int32).reshape(B, 1, N)

    nc = 32768
    nchunks = N // nc

    out = pl.pallas_call(
        functools.partial(_body, nchunks),
        grid=(B, 2, nchunks),
        in_specs=[
            pl.BlockSpec((1, E, nc), lambda b, p, n: (b, 0, n)),
            pl.BlockSpec((1, 1, nc), lambda b, p, n: (b, 0, n)),
        ],
        out_specs=pl.BlockSpec((1, 4), lambda b, p, n: (0, 0)),
        out_shape=jax.ShapeDtypeStruct((1, 4), jnp.float32),
        scratch_shapes=[
            pltpu.VMEM((_C, E), jnp.float32),
            pltpu.VMEM((_C, E), jnp.float32),
            pltpu.VMEM((_C, 1), jnp.float32),
            pltpu.VMEM((_C, 1), jnp.float32),
            pltpu.VMEM((_C, 1), jnp.float32),
            pltpu.SMEM((4,), jnp.float32),
        ],
    )(emb3, mask3)
    return out[0, 0], out[0, 1], out[0, 2], out[0, 3]


# speedup vs baseline: 19.8247x; 1.0664x over previous
import functools

import jax
import jax.numpy as jnp
from jax.experimental import pallas as pl
from jax.experimental.pallas import tpu as pltpu

_DELTA_VAR = 0.5
_DELTA_DIST = 1.5
_ALPHA = 1.0
_BETA = 1.0
_GAMMA = 0.001
_C = 32
_EPS = 1e-12

_HI = jax.lax.Precision.HIGHEST


def _body(nchunks, emb_ref, mask_ref, out_ref,
          sums_ref, cent_ref, cn2r_ref, cntr_ref, hsr_ref, acc_ref):
    b = pl.program_id(0)
    ph = pl.program_id(1)
    n = pl.program_id(2)
    nb = pl.num_programs(0)

    x = emb_ref[0]
    m = mask_ref[0]
    C = _C
    E = x.shape[0]
    Nc = x.shape[1]
    iota_c = jax.lax.broadcasted_iota(jnp.int32, (C, Nc), 0)
    oh = (m == iota_c).astype(jnp.float32)

    @pl.when((b == 0) & (ph == 0) & (n == 0))
    def _init_acc():
        acc_ref[0] = 0.0
        acc_ref[1] = 0.0
        acc_ref[2] = 0.0
        acc_ref[3] = 0.0

    @pl.when(ph == 0)
    def _phase0():
        @pl.when(n == 0)
        def _z():
            sums_ref[...] = jnp.zeros_like(sums_ref)
            cntr_ref[...] = jnp.zeros_like(cntr_ref)

        ones_r = jnp.ones((1, Nc), jnp.float32)
        sums_ref[...] += jax.lax.dot_general(
            x, oh, (((1,), (1,)), ((), ())))
        cntr_ref[...] += jax.lax.dot_general(
            ones_r, oh, (((1,), (1,)), ((), ())))

        @pl.when(n == nchunks - 1)
        def _centers():
            safe = jnp.maximum(cntr_ref[...], 1.0)
            cent = sums_ref[...] / safe
            cent_ref[...] = cent
            cn2r_ref[...] = jnp.sum(cent * cent, axis=0, keepdims=True)

    @pl.when(ph == 1)
    def _phase1():
        @pl.when(n == 0)
        def _z():
            hsr_ref[...] = jnp.zeros_like(hsr_ref)

        cent = cent_ref[...]
        cpp = jax.lax.dot_general(
            cent, oh, (((1,), (0,)), ((), ())))
        cn2pp = jax.lax.dot_general(
            cn2r_ref[...], oh, (((1,), (0,)), ((), ())))
        u = x * (x - 2.0 * cpp)
        ones_e = jnp.ones((1, E), jnp.float32)
        d2 = jax.lax.dot_general(
            ones_e, u, (((1,), (0,)), ((), ()))) + cn2pp
        d = jnp.sqrt(jnp.maximum(d2, 0.0) + _EPS)
        hinged = jnp.maximum(d - _DELTA_VAR, 0.0) ** 2
        hsr_ref[...] += jax.lax.dot_general(
            hinged, oh, (((1,), (1,)), ((), ())))

        @pl.when(n == nchunks - 1)
        def _finish():
            cnt = cntr_ref[...]
            safe = jnp.maximum(cnt, 1.0)
            per_inst = hsr_ref[...] / safe
            ids_r = jax.lax.broadcasted_iota(jnp.int32, (1, C), 1)
            validr = ((cnt > 0.0) & (ids_r > 0)).astype(jnp.float32)
            n_inst = jnp.sum(validr)
            lv_b = jnp.sum(validr * per_inst) / jnp.maximum(n_inst, 1.0)

            cent = cent_ref[...]
            g = jax.lax.dot_general(
                cent, cent, (((0,), (0,)), ((), ())), precision=_HI)
            i0 = jax.lax.broadcasted_iota(jnp.int32, (C, C), 0)
            i1 = jax.lax.broadcasted_iota(jnp.int32, (C, C), 1)
            eye = (i0 == i1).astype(jnp.float32)
            cn2r = cn2r_ref[...]
            cn2c = jnp.sum(g * eye, axis=1, keepdims=True)
            dist2 = jnp.maximum(cn2c + cn2r - 2.0 * g, 0.0)
            dist = jnp.sqrt(dist2 + _EPS)
            validc = jnp.sum(eye * validr, axis=1, keepdims=True)
            pairm = validc * validr * (i0 < i1).astype(jnp.float32)
            hd = jnp.maximum(2.0 * _DELTA_DIST - dist, 0.0) ** 2
            npairs = jnp.sum(pairm)
            ld_b = jnp.sum(pairm * hd) / jnp.maximum(npairs, 1.0)

            norms = jnp.sqrt(cn2r + _EPS)
            lr_b = jnp.sum(validr * norms) / jnp.maximum(n_inst, 1.0)

            has = (n_inst > 0.0).astype(jnp.float32)
            acc_ref[0] += lv_b * has
            acc_ref[1] += ld_b * has
            acc_ref[2] += lr_b * has
            acc_ref[3] += has

            @pl.when(b == nb - 1)
            def _emit():
                denom = jnp.maximum(acc_ref[3], 1.0)
                lv = acc_ref[0] / denom
                ld = acc_ref[1] / denom
                lr = acc_ref[2] / denom
                tot = _ALPHA * lv + _BETA * ld + _GAMMA * lr
                lane = jax.lax.broadcasted_iota(jnp.int32, (1, 4), 1)
                out_ref[...] = (
                    tot * (lane == 0) + lv * (lane == 1)
                    + ld * (lane == 2) + lr * (lane == 3)
                ).astype(jnp.float32)


@jax.jit
def kernel(embedding, instance_mask):
    B, E = embedding.shape[0], embedding.shape[1]
    N = embedding.shape[2] * embedding.shape[3]
    emb3 = embedding.reshape(B, E, N)
    mask3 = instance_mask.astype(jnp.int32).reshape(B, 1, N)

    nc = 32768
    nchunks = N // nc

    out = pl.pallas_call(
        functools.partial(_body, nchunks),
        grid=(B, 2, nchunks),
        in_specs=[
            pl.BlockSpec((1, E, nc), lambda b, p, n: (b, 0, n)),
            pl.BlockSpec((1, 1, nc), lambda b, p, n: (b, 0, n)),
        ],
        out_specs=pl.BlockSpec((1, 4), lambda b, p, n: (0, 0)),
        out_shape=jax.ShapeDtypeStruct((1, 4), jnp.float32),
        scratch_shapes=[
            pltpu.VMEM((E, _C), jnp.float32),
            pltpu.VMEM((E, _C), jnp.float32),
            pltpu.VMEM((1, _C), jnp.float32),
            pltpu.VMEM((1, _C), jnp.float32),
            pltpu.VMEM((1, _C), jnp.float32),
            pltpu.SMEM((4,), jnp.float32),
        ],
    )(emb3, mask3)
    return out[0, 0], out[0, 1], out[0, 2], out[0, 3]


# speedup vs baseline: 19.9558x; 1.0066x over previous
import functools

import jax
import jax.numpy as jnp
from jax.experimental import pallas as pl
from jax.experimental.pallas import tpu as pltpu

_DELTA_VAR = 0.5
_DELTA_DIST = 1.5
_ALPHA = 1.0
_BETA = 1.0
_GAMMA = 0.001
_C = 32
_EPS = 1e-12

_HI = jax.lax.Precision.HIGHEST


def _body(nchunks, emb_ref, mask_ref, out_ref,
          sums_ref, cent_ref, cn2r_ref, cntr_ref, hsr_ref, acc_ref):
    b = pl.program_id(0)
    ph = pl.program_id(1)
    n = pl.program_id(2)
    nb = pl.num_programs(0)

    x = emb_ref[0]
    m = mask_ref[0]
    C = _C
    E = x.shape[0]
    Nc = x.shape[1]
    iota_c = jax.lax.broadcasted_iota(jnp.int32, (C, Nc), 0)
    oh = (m == iota_c).astype(jnp.float32)

    @pl.when((b == 0) & (ph == 0) & (n == 0))
    def _init_acc():
        acc_ref[0] = 0.0
        acc_ref[1] = 0.0
        acc_ref[2] = 0.0
        acc_ref[3] = 0.0

    @pl.when(ph == 0)
    def _phase0():
        @pl.when(n == 0)
        def _z():
            sums_ref[...] = jnp.zeros_like(sums_ref)
            cntr_ref[...] = jnp.zeros_like(cntr_ref)

        ones_r = jnp.ones((1, Nc), jnp.float32)
        sums_ref[...] += jax.lax.dot_general(
            x, oh, (((1,), (1,)), ((), ())))
        cntr_ref[...] += jax.lax.dot_general(
            ones_r, oh, (((1,), (1,)), ((), ())))

        @pl.when(n == nchunks - 1)
        def _centers():
            safe = jnp.maximum(cntr_ref[...], 1.0)
            cent = sums_ref[...] / safe
            cent_ref[...] = cent
            cn2r_ref[...] = jnp.sum(cent * cent, axis=0, keepdims=True)

    @pl.when(ph == 1)
    def _phase1():
        @pl.when(n == 0)
        def _z():
            hsr_ref[...] = jnp.zeros_like(hsr_ref)

        cent = cent_ref[...]
        cpp = jax.lax.dot_general(
            cent, oh, (((1,), (0,)), ((), ())))
        cn2pp = jax.lax.dot_general(
            cn2r_ref[...], oh, (((1,), (0,)), ((), ())))
        u = x * (x - 2.0 * cpp)
        ones_e = jnp.ones((1, E), jnp.float32)
        d2 = jax.lax.dot_general(
            ones_e, u, (((1,), (0,)), ((), ()))) + cn2pp
        d = jnp.sqrt(jnp.maximum(d2, 0.0) + _EPS)
        hinged = jnp.maximum(d - _DELTA_VAR, 0.0) ** 2
        hsr_ref[...] += jax.lax.dot_general(
            hinged, oh, (((1,), (1,)), ((), ())))

        @pl.when(n == nchunks - 1)
        def _finish():
            cnt = cntr_ref[...]
            safe = jnp.maximum(cnt, 1.0)
            per_inst = hsr_ref[...] / safe
            ids_r = jax.lax.broadcasted_iota(jnp.int32, (1, C), 1)
            validr = ((cnt > 0.0) & (ids_r > 0)).astype(jnp.float32)
            n_inst = jnp.sum(validr)
            lv_b = jnp.sum(validr * per_inst) / jnp.maximum(n_inst, 1.0)

            cent = cent_ref[...]
            g = jax.lax.dot_general(
                cent, cent, (((0,), (0,)), ((), ())), precision=_HI)
            i0 = jax.lax.broadcasted_iota(jnp.int32, (C, C), 0)
            i1 = jax.lax.broadcasted_iota(jnp.int32, (C, C), 1)
            eye = (i0 == i1).astype(jnp.float32)
            cn2r = cn2r_ref[...]
            cn2c = jnp.sum(g * eye, axis=1, keepdims=True)
            dist2 = jnp.maximum(cn2c + cn2r - 2.0 * g, 0.0)
            dist = jnp.sqrt(dist2 + _EPS)
            validc = jnp.sum(eye * validr, axis=1, keepdims=True)
            pairm = validc * validr * (i0 < i1).astype(jnp.float32)
            hd = jnp.maximum(2.0 * _DELTA_DIST - dist, 0.0) ** 2
            npairs = jnp.sum(pairm)
            ld_b = jnp.sum(pairm * hd) / jnp.maximum(npairs, 1.0)

            norms = jnp.sqrt(cn2r + _EPS)
            lr_b = jnp.sum(validr * norms) / jnp.maximum(n_inst, 1.0)

            has = (n_inst > 0.0).astype(jnp.float32)
            acc_ref[0] += lv_b * has
            acc_ref[1] += ld_b * has
            acc_ref[2] += lr_b * has
            acc_ref[3] += has

            @pl.when(b == nb - 1)
            def _emit():
                denom = jnp.maximum(acc_ref[3], 1.0)
                lv = acc_ref[0] / denom
                ld = acc_ref[1] / denom
                lr = acc_ref[2] / denom
                tot = _ALPHA * lv + _BETA * ld + _GAMMA * lr
                lane = jax.lax.broadcasted_iota(jnp.int32, (1, 4), 1)
                out_ref[...] = (
                    tot * (lane == 0) + lv * (lane == 1)
                    + ld * (lane == 2) + lr * (lane == 3)
                ).astype(jnp.float32)


@jax.jit
def kernel(embedding, instance_mask):
    B, E = embedding.shape[0], embedding.shape[1]
    N = embedding.shape[2] * embedding.shape[3]
    emb3 = embedding.reshape(B, E, N)
    mask3 = instance_mask.astype(jnp.int32).reshape(B, 1, N)

    nc = 65536
    nchunks = N // nc

    out = pl.pallas_call(
        functools.partial(_body, nchunks),
        grid=(B, 2, nchunks),
        in_specs=[
            pl.BlockSpec((1, E, nc), lambda b, p, n: (b, 0, n)),
            pl.BlockSpec((1, 1, nc), lambda b, p, n: (b, 0, n)),
        ],
        out_specs=pl.BlockSpec((1, 4), lambda b, p, n: (0, 0)),
        out_shape=jax.ShapeDtypeStruct((1, 4), jnp.float32),
        scratch_shapes=[
            pltpu.VMEM((E, _C), jnp.float32),
            pltpu.VMEM((E, _C), jnp.float32),
            pltpu.VMEM((1, _C), jnp.float32),
            pltpu.VMEM((1, _C), jnp.float32),
            pltpu.VMEM((1, _C), jnp.float32),
            pltpu.SMEM((4,), jnp.float32),
        ],
    )(emb3, mask3)
    return out[0, 0], out[0, 1], out[0, 2], out[0, 3]


# speedup vs baseline: 20.2288x; 1.0137x over previous
import functools

import jax
import jax.numpy as jnp
from jax.experimental import pallas as pl
from jax.experimental.pallas import tpu as pltpu

_DELTA_VAR = 0.5
_DELTA_DIST = 1.5
_ALPHA = 1.0
_BETA = 1.0
_GAMMA = 0.001
_C = 32
_EPS = 1e-12

_HI = jax.lax.Precision.HIGHEST


def _body(nchunks, nc, emb_ref, mask_ref, out_ref,
          buf_ref, sums_ref, cent_ref, cn2r_ref, cntr_ref, hsr_ref, acc_ref,
          sem_ref):
    b = pl.program_id(0)
    ph = pl.program_id(1)
    n = pl.program_id(2)
    nb = pl.num_programs(0)

    C = _C
    E = buf_ref.shape[0]

    def chunk_copy(bb, k):
        return pltpu.make_async_copy(
            emb_ref.at[bb, :, pl.ds(k * nc, nc)],
            buf_ref.at[:, pl.ds(k * nc, nc)],
            sem_ref.at[k])

    @pl.when((b == 0) & (ph == 0) & (n == 0))
    def _prologue():
        acc_ref[0] = 0.0
        acc_ref[1] = 0.0
        acc_ref[2] = 0.0
        acc_ref[3] = 0.0
        for k in range(nchunks):
            chunk_copy(0, k).start()

    m = mask_ref[0]
    iota_c = jax.lax.broadcasted_iota(jnp.int32, (C, nc), 0)
    oh = (m == iota_c).astype(jnp.float32)

    @pl.when(ph == 0)
    def _phase0():
        @pl.when(n == 0)
        def _z():
            sums_ref[...] = jnp.zeros_like(sums_ref)
            cntr_ref[...] = jnp.zeros_like(cntr_ref)

        chunk_copy(b, n).wait()
        x = buf_ref[:, pl.ds(n * nc, nc)]
        ones_r = jnp.ones((1, nc), jnp.float32)
        sums_ref[...] += jax.lax.dot_general(
            x, oh, (((1,), (1,)), ((), ())))
        cntr_ref[...] += jax.lax.dot_general(
            ones_r, oh, (((1,), (1,)), ((), ())))

        @pl.when(n == nchunks - 1)
        def _centers():
            safe = jnp.maximum(cntr_ref[...], 1.0)
            cent = sums_ref[...] / safe
            cent_ref[...] = cent
            cn2r_ref[...] = jnp.sum(cent * cent, axis=0, keepdims=True)

    @pl.when(ph == 1)
    def _phase1():
        @pl.when(n == 0)
        def _z():
            hsr_ref[...] = jnp.zeros_like(hsr_ref)

        x = buf_ref[:, pl.ds(n * nc, nc)]
        cent = cent_ref[...]
        cpp = jax.lax.dot_general(
            cent, oh, (((1,), (0,)), ((), ())))
        cn2pp = jax.lax.dot_general(
            cn2r_ref[...], oh, (((1,), (0,)), ((), ())))
        u = x * (x - 2.0 * cpp)
        ones_e = jnp.ones((1, E), jnp.float32)
        d2 = jax.lax.dot_general(
            ones_e, u, (((1,), (0,)), ((), ()))) + cn2pp
        d = jnp.sqrt(jnp.maximum(d2, 0.0) + _EPS)
        hinged = jnp.maximum(d - _DELTA_VAR, 0.0) ** 2
        hsr_ref[...] += jax.lax.dot_general(
            hinged, oh, (((1,), (1,)), ((), ())))

        @pl.when(b < nb - 1)
        def _prefetch_next():
            chunk_copy(b + 1, n).start()

        @pl.when(n == nchunks - 1)
        def _finish():
            cnt = cntr_ref[...]
            safe = jnp.maximum(cnt, 1.0)
            per_inst = hsr_ref[...] / safe
            ids_r = jax.lax.broadcasted_iota(jnp.int32, (1, C), 1)
            validr = ((cnt > 0.0) & (ids_r > 0)).astype(jnp.float32)
            n_inst = jnp.sum(validr)
            lv_b = jnp.sum(validr * per_inst) / jnp.maximum(n_inst, 1.0)

            cent = cent_ref[...]
            g = jax.lax.dot_general(
                cent, cent, (((0,), (0,)), ((), ())), precision=_HI)
            i0 = jax.lax.broadcasted_iota(jnp.int32, (C, C), 0)
            i1 = jax.lax.broadcasted_iota(jnp.int32, (C, C), 1)
            eye = (i0 == i1).astype(jnp.float32)
            cn2r = cn2r_ref[...]
            cn2c = jnp.sum(g * eye, axis=1, keepdims=True)
            dist2 = jnp.maximum(cn2c + cn2r - 2.0 * g, 0.0)
            dist = jnp.sqrt(dist2 + _EPS)
            validc = jnp.sum(eye * validr, axis=1, keepdims=True)
            pairm = validc * validr * (i0 < i1).astype(jnp.float32)
            hd = jnp.maximum(2.0 * _DELTA_DIST - dist, 0.0) ** 2
            npairs = jnp.sum(pairm)
            ld_b = jnp.sum(pairm * hd) / jnp.maximum(npairs, 1.0)

            norms = jnp.sqrt(cn2r + _EPS)
            lr_b = jnp.sum(validr * norms) / jnp.maximum(n_inst, 1.0)

            has = (n_inst > 0.0).astype(jnp.float32)
            acc_ref[0] += lv_b * has
            acc_ref[1] += ld_b * has
            acc_ref[2] += lr_b * has
            acc_ref[3] += has

            @pl.when(b == nb - 1)
            def _emit():
                denom = jnp.maximum(acc_ref[3], 1.0)
                lv = acc_ref[0] / denom
                ld = acc_ref[1] / denom
                lr = acc_ref[2] / denom
                tot = _ALPHA * lv + _BETA * ld + _GAMMA * lr
                lane = jax.lax.broadcasted_iota(jnp.int32, (1, 4), 1)
                out_ref[...] = (
                    tot * (lane == 0) + lv * (lane == 1)
                    + ld * (lane == 2) + lr * (lane == 3)
                ).astype(jnp.float32)


@jax.jit
def kernel(embedding, instance_mask):
    B, E = embedding.shape[0], embedding.shape[1]
    N = embedding.shape[2] * embedding.shape[3]
    emb3 = embedding.reshape(B, E, N)
    mask3 = instance_mask.astype(jnp.int32).reshape(B, 1, N)

    nc = 16384
    nchunks = N // nc

    out = pl.pallas_call(
        functools.partial(_body, nchunks, nc),
        grid=(B, 2, nchunks),
        in_specs=[
            pl.BlockSpec(memory_space=pl.ANY),
            pl.BlockSpec((1, 1, nc), lambda b, p, n: (b, 0, n)),
        ],
        out_specs=pl.BlockSpec((1, 4), lambda b, p, n: (0, 0)),
        out_shape=jax.ShapeDtypeStruct((1, 4), jnp.float32),
        scratch_shapes=[
            pltpu.VMEM((E, N), jnp.float32),
            pltpu.VMEM((E, _C), jnp.float32),
            pltpu.VMEM((E, _C), jnp.float32),
            pltpu.VMEM((1, _C), jnp.float32),
            pltpu.VMEM((1, _C), jnp.float32),
            pltpu.VMEM((1, _C), jnp.float32),
            pltpu.SMEM((4,), jnp.float32),
            pltpu.SemaphoreType.DMA((N // 16384,)),
        ],
    )(emb3, mask3)
    return out[0, 0], out[0, 1], out[0, 2], out[0, 3]


# speedup vs baseline: 21.0710x; 1.0416x over previous
import functools

import jax
import jax.numpy as jnp
from jax.experimental import pallas as pl
from jax.experimental.pallas import tpu as pltpu

_DELTA_VAR = 0.5
_DELTA_DIST = 1.5
_ALPHA = 1.0
_BETA = 1.0
_GAMMA = 0.001
_C = 32
_EPS = 1e-12

_HI = jax.lax.Precision.HIGHEST


def _body(nchunks, nc, emb_ref, mask_ref, out_ref,
          buf_ref, sums_ref, cent_ref, cn2r_ref, cntr_ref, hsr_ref, acc_ref,
          sem_ref):
    b = pl.program_id(0)
    ph = pl.program_id(1)
    n = pl.program_id(2)
    nb = pl.num_programs(0)

    C = _C
    E = buf_ref.shape[0]

    def chunk_copy(bb, k):
        return pltpu.make_async_copy(
            emb_ref.at[bb, :, pl.ds(k * nc, nc)],
            buf_ref.at[:, pl.ds(k * nc, nc)],
            sem_ref.at[k])

    @pl.when((b == 0) & (ph == 0) & (n == 0))
    def _prologue():
        acc_ref[0] = 0.0
        acc_ref[1] = 0.0
        acc_ref[2] = 0.0
        acc_ref[3] = 0.0
        for k in range(nchunks):
            chunk_copy(0, k).start()

    m = mask_ref[0]
    iota_c = jax.lax.broadcasted_iota(jnp.int32, (C, nc), 0)
    oh = (m == iota_c).astype(jnp.float32)

    @pl.when(ph == 0)
    def _phase0():
        @pl.when(n == 0)
        def _z():
            sums_ref[...] = jnp.zeros_like(sums_ref)
            cntr_ref[...] = jnp.zeros_like(cntr_ref)

        chunk_copy(b, n).wait()
        x = buf_ref[:, pl.ds(n * nc, nc)]
        ones_r = jnp.ones((1, nc), jnp.float32)
        sums_ref[...] += jax.lax.dot_general(
            x, oh, (((1,), (1,)), ((), ())))
        cntr_ref[...] += jax.lax.dot_general(
            ones_r, oh, (((1,), (1,)), ((), ())))

        @pl.when(n == nchunks - 1)
        def _centers():
            safe = jnp.maximum(cntr_ref[...], 1.0)
            cent = sums_ref[...] / safe
            cent_ref[...] = cent
            cn2r_ref[...] = jnp.sum(cent * cent, axis=0, keepdims=True)

    @pl.when(ph == 1)
    def _phase1():
        @pl.when(n == 0)
        def _z():
            hsr_ref[...] = jnp.zeros_like(hsr_ref)

        x = buf_ref[:, pl.ds(n * nc, nc)]
        cent = cent_ref[...]
        cpp = jax.lax.dot_general(
            cent, oh, (((1,), (0,)), ((), ())))
        cn2pp = jax.lax.dot_general(
            cn2r_ref[...], oh, (((1,), (0,)), ((), ())))
        u = x * (x - 2.0 * cpp)
        ones_e = jnp.ones((1, E), jnp.float32)
        d2 = jax.lax.dot_general(
            ones_e, u, (((1,), (0,)), ((), ()))) + cn2pp
        d = jnp.sqrt(jnp.maximum(d2, 0.0) + _EPS)
        hinged = jnp.maximum(d - _DELTA_VAR, 0.0) ** 2
        hsr_ref[...] += jax.lax.dot_general(
            hinged, oh, (((1,), (1,)), ((), ())))

        @pl.when(b < nb - 1)
        def _prefetch_next():
            chunk_copy(b + 1, n).start()

        @pl.when(n == nchunks - 1)
        def _finish():
            cnt = cntr_ref[...]
            safe = jnp.maximum(cnt, 1.0)
            per_inst = hsr_ref[...] / safe
            ids_r = jax.lax.broadcasted_iota(jnp.int32, (1, C), 1)
            validr = ((cnt > 0.0) & (ids_r > 0)).astype(jnp.float32)
            n_inst = jnp.sum(validr)
            lv_b = jnp.sum(validr * per_inst) / jnp.maximum(n_inst, 1.0)

            cent = cent_ref[...]
            g = jax.lax.dot_general(
                cent, cent, (((0,), (0,)), ((), ())), precision=_HI)
            i0 = jax.lax.broadcasted_iota(jnp.int32, (C, C), 0)
            i1 = jax.lax.broadcasted_iota(jnp.int32, (C, C), 1)
            eye = (i0 == i1).astype(jnp.float32)
            cn2r = cn2r_ref[...]
            cn2c = jnp.sum(g * eye, axis=1, keepdims=True)
            dist2 = jnp.maximum(cn2c + cn2r - 2.0 * g, 0.0)
            dist = jnp.sqrt(dist2 + _EPS)
            validc = jnp.sum(eye * validr, axis=1, keepdims=True)
            pairm = validc * validr * (i0 < i1).astype(jnp.float32)
            hd = jnp.maximum(2.0 * _DELTA_DIST - dist, 0.0) ** 2
            npairs = jnp.sum(pairm)
            ld_b = jnp.sum(pairm * hd) / jnp.maximum(npairs, 1.0)

            norms = jnp.sqrt(cn2r + _EPS)
            lr_b = jnp.sum(validr * norms) / jnp.maximum(n_inst, 1.0)

            has = (n_inst > 0.0).astype(jnp.float32)
            acc_ref[0] += lv_b * has
            acc_ref[1] += ld_b * has
            acc_ref[2] += lr_b * has
            acc_ref[3] += has

            @pl.when(b == nb - 1)
            def _emit():
                denom = jnp.maximum(acc_ref[3], 1.0)
                lv = acc_ref[0] / denom
                ld = acc_ref[1] / denom
                lr = acc_ref[2] / denom
                tot = _ALPHA * lv + _BETA * ld + _GAMMA * lr
                lane = jax.lax.broadcasted_iota(jnp.int32, (1, 4), 1)
                out_ref[...] = (
                    tot * (lane == 0) + lv * (lane == 1)
                    + ld * (lane == 2) + lr * (lane == 3)
                ).astype(jnp.float32)


@jax.jit
def kernel(embedding, instance_mask):
    B, E = embedding.shape[0], embedding.shape[1]
    N = embedding.shape[2] * embedding.shape[3]
    emb3 = embedding.reshape(B, E, N)
    mask3 = instance_mask.astype(jnp.int32).reshape(B, 1, N)

    nc = 32768
    nchunks = N // nc

    out = pl.pallas_call(
        functools.partial(_body, nchunks, nc),
        grid=(B, 2, nchunks),
        in_specs=[
            pl.BlockSpec(memory_space=pl.ANY),
            pl.BlockSpec((1, 1, nc), lambda b, p, n: (b, 0, n)),
        ],
        out_specs=pl.BlockSpec((1, 4), lambda b, p, n: (0, 0)),
        out_shape=jax.ShapeDtypeStruct((1, 4), jnp.float32),
        scratch_shapes=[
            pltpu.VMEM((E, N), jnp.float32),
            pltpu.VMEM((E, _C), jnp.float32),
            pltpu.VMEM((E, _C), jnp.float32),
            pltpu.VMEM((1, _C), jnp.float32),
            pltpu.VMEM((1, _C), jnp.float32),
            pltpu.VMEM((1, _C), jnp.float32),
            pltpu.SMEM((4,), jnp.float32),
            pltpu.SemaphoreType.DMA((N // 32768,)),
        ],
    )(emb3, mask3)
    return out[0, 0], out[0, 1], out[0, 2], out[0, 3]
